# Initial kernel scaffold; baseline (speedup 1.0000x reference)
import jax
import jax.numpy as jnp
from jax.experimental import pallas as pl


def kernel(x, edge_index, batch, W1, b1, gamma, beta, W2, b2, Wlin, blin):
    raise NotImplementedError("write your pallas kernel here")


# speedup vs baseline: 9.3253x; 9.3253x over previous
import functools

import jax
import jax.numpy as jnp
from jax import lax
from jax.experimental import pallas as pl
from jax.experimental.pallas import tpu as pltpu
from jax.experimental.pallas import tpu_sc as plsc

_N = 10000
_E = 320000
_F = 128
_G = 64

_NC = 2
_NS = 16
_NW = _NC * _NS
_CH = 128
_CPW = 80
_NP = 10240
_RPT = _NP // _NS

_BLK = 1000
_NB = _N // _BLK


def _vector_mesh():
    return plsc.VectorSubcoreMesh(core_axis_name="c", subcore_axis_name="s")


def _sc_deg(dst_r, ones128, zeros128):
    @functools.partial(
        pl.kernel,
        out_type=jax.ShapeDtypeStruct((_NC, _NP, _F), jnp.float32),
        mesh=_vector_mesh(),
        scratch_types=[
            pltpu.VMEM((_CPW, _CH), jnp.int32),
            pltpu.VMEM((_CH, _F), jnp.float32),
            pltpu.VMEM_SHARED((_NP, _F), jnp.float32),
        ],
    )
    def k(dst_hbm, ones_hbm, z_hbm, out_hbm, didx, ones_v, acc_sh):
        cid = lax.axis_index("c")
        sid = lax.axis_index("s")
        w = sid * _NC + cid
        pltpu.sync_copy(ones_hbm, ones_v)
        pltpu.sync_copy(dst_hbm.at[w], didx)
        pltpu.sync_copy(z_hbm, acc_sh.at[pl.ds(sid * _RPT, _RPT)])
        plsc.subcore_barrier()

        @pl.loop(0, _CPW)
        def _(j):
            pltpu.sync_copy(ones_v, acc_sh.at[didx.at[j]], add=True)

        plsc.subcore_barrier()
        pltpu.sync_copy(
            acc_sh.at[pl.ds(sid * _RPT, _RPT)],
            out_hbm.at[cid, pl.ds(sid * _RPT, _RPT)],
        )

    return k(dst_r, ones128, zeros128)


def _sc_msg(hs, src_r, dst_r, zeros128):
    @functools.partial(
        pl.kernel,
        out_type=jax.ShapeDtypeStruct((_NC, _NP, _F), jnp.float32),
        mesh=_vector_mesh(),
        scratch_types=[
            pltpu.VMEM((_CPW, _CH), jnp.int32),
            pltpu.VMEM((_CPW, _CH), jnp.int32),
            pltpu.VMEM((_CH, _F), jnp.float32),
            pltpu.VMEM_SHARED((_NP, _F), jnp.float32),
        ],
    )
    def k(hs_hbm, src_hbm, dst_hbm, z_hbm, out_hbm,
          sidx, didx, rows, acc_sh):
        cid = lax.axis_index("c")
        sid = lax.axis_index("s")
        w = sid * _NC + cid
        pltpu.sync_copy(src_hbm.at[w], sidx)
        pltpu.sync_copy(dst_hbm.at[w], didx)
        pltpu.sync_copy(z_hbm, acc_sh.at[pl.ds(sid * _RPT, _RPT)])
        plsc.subcore_barrier()

        @pl.loop(0, _CPW)
        def _(j):
            pltpu.sync_copy(hs_hbm.at[sidx.at[j]], rows)
            pltpu.sync_copy(rows, acc_sh.at[didx.at[j]], add=True)

        plsc.subcore_barrier()
        pltpu.sync_copy(
            acc_sh.at[pl.ds(sid * _RPT, _RPT)],
            out_hbm.at[cid, pl.ds(sid * _RPT, _RPT)],
        )

    return k(hs, src_r, dst_r, zeros128)


def _tc_matmul(x, W):
    n, f = x.shape
    c = W.shape[1]

    def body(x_ref, w_ref, o_ref):
        o_ref[...] = jnp.dot(x_ref[...], w_ref[...],
                             preferred_element_type=jnp.float32)

    return pl.pallas_call(
        body,
        grid=(n // _BLK,),
        in_specs=[
            pl.BlockSpec((_BLK, f), lambda i: (i, 0)),
            pl.BlockSpec((f, c), lambda i: (0, 0)),
        ],
        out_specs=pl.BlockSpec((_BLK, c), lambda i: (i, 0)),
        out_shape=jax.ShapeDtypeStruct((n, c), jnp.float32),
    )(x, W)


def _tc_dis_prescale(degp, h0):
    def body(d_ref, h_ref, dis_ref, hs_ref):
        deg = d_ref[0, :, 0:1] + d_ref[1, :, 0:1] + 1.0
        dis = lax.rsqrt(deg)
        dis_ref[...] = dis
        hs_ref[...] = h_ref[...] * dis

    return pl.pallas_call(
        body,
        grid=(_NB,),
        in_specs=[
            pl.BlockSpec((2, _BLK, _F), lambda i: (0, i, 0)),
            pl.BlockSpec((_BLK, _F), lambda i: (i, 0)),
        ],
        out_specs=[
            pl.BlockSpec((_BLK, 1), lambda i: (i, 0)),
            pl.BlockSpec((_BLK, _F), lambda i: (i, 0)),
        ],
        out_shape=[
            jax.ShapeDtypeStruct((_N, 1), jnp.float32),
            jax.ShapeDtypeStruct((_NP, _F), jnp.float32),
        ],
    )(degp, h0)


def _tc_layer1_post(part, hs1, dis, b1r):
    def body(p_ref, hs_ref, dis_ref, b_ref, a_ref, st_ref):
        i = pl.program_id(0)
        z = dis_ref[...] * (p_ref[0] + p_ref[1] + hs_ref[...]) + b_ref[...]
        a = jnp.maximum(z, 0.0)
        a_ref[...] = a
        s = jnp.sum(a, axis=0, keepdims=True)
        ss = jnp.sum(a * a, axis=0, keepdims=True)
        st = jnp.concatenate([s, ss], axis=0)

        @pl.when(i == 0)
        def _():
            st_ref[...] = st

        @pl.when(i > 0)
        def _():
            st_ref[...] = st_ref[...] + st

    return pl.pallas_call(
        body,
        grid=(_NB,),
        in_specs=[
            pl.BlockSpec((2, _BLK, _F), lambda i: (0, i, 0)),
            pl.BlockSpec((_BLK, _F), lambda i: (i, 0)),
            pl.BlockSpec((_BLK, 1), lambda i: (i, 0)),
            pl.BlockSpec((1, _F), lambda i: (0, 0)),
        ],
        out_specs=[
            pl.BlockSpec((_BLK, _F), lambda i: (i, 0)),
            pl.BlockSpec((2, _F), lambda i: (0, 0)),
        ],
        out_shape=[
            jax.ShapeDtypeStruct((_N, _F), jnp.float32),
            jax.ShapeDtypeStruct((2, _F), jnp.float32),
        ],
    )(part, hs1, dis, b1r)


def _tc_bn_mm(a, stats, gr, br, dis, W2):
    def body(a_ref, st_ref, g_ref, be_ref, dis_ref, w_ref, o_ref):
        mean = st_ref[0:1, :] * (1.0 / _N)
        var = st_ref[1:2, :] * (1.0 / _N) - mean * mean
        inv = lax.rsqrt(var + 1e-5)
        h1 = (a_ref[...] - mean) * (inv * g_ref[...]) + be_ref[...]
        g = jnp.dot(h1, w_ref[...], preferred_element_type=jnp.float32)
        o_ref[...] = g * dis_ref[...]

    return pl.pallas_call(
        body,
        grid=(_NB,),
        in_specs=[
            pl.BlockSpec((_BLK, _F), lambda i: (i, 0)),
            pl.BlockSpec((2, _F), lambda i: (0, 0)),
            pl.BlockSpec((1, _F), lambda i: (0, 0)),
            pl.BlockSpec((1, _F), lambda i: (0, 0)),
            pl.BlockSpec((_BLK, 1), lambda i: (i, 0)),
            pl.BlockSpec((_F, _F), lambda i: (0, 0)),
        ],
        out_specs=pl.BlockSpec((_BLK, _F), lambda i: (i, 0)),
        out_shape=jax.ShapeDtypeStruct((_NP, _F), jnp.float32),
    )(a, stats, gr, br, dis, W2)


def _tc_final(part, hs2, dis, b2r, batch3, Wlin, blr):
    def body(p_ref, hs_ref, dis_ref, b_ref, bat_ref, wl_ref, bl_ref,
             o_ref, pool_ref, cnt_ref):
        i = pl.program_id(0)
        h2 = dis_ref[...] * (p_ref[0] + p_ref[1] + hs_ref[...]) + b_ref[...]
        seg = lax.broadcasted_iota(jnp.int32, (_G, _BLK), 0)
        oht = (bat_ref[0] == seg).astype(jnp.float32)
        psum = lax.dot_general(oht, h2, (((1,), (0,)), ((), ())),
                               preferred_element_type=jnp.float32)
        csum = lax.dot_general(oht, jnp.ones_like(h2), (((1,), (0,)), ((), ())),
                               preferred_element_type=jnp.float32)

        @pl.when(i == 0)
        def _():
            pool_ref[...] = psum
            cnt_ref[...] = csum

        @pl.when(i > 0)
        def _():
            pool_ref[...] = pool_ref[...] + psum
            cnt_ref[...] = cnt_ref[...] + csum

        @pl.when(i == _NB - 1)
        def _():
            pooled = pool_ref[...] / jnp.maximum(cnt_ref[...], 1.0)
            o_ref[...] = jnp.dot(pooled, wl_ref[...],
                                 preferred_element_type=jnp.float32) + bl_ref[...]

    return pl.pallas_call(
        body,
        grid=(_NB,),
        in_specs=[
            pl.BlockSpec((2, _BLK, _F), lambda i: (0, i, 0)),
            pl.BlockSpec((_BLK, _F), lambda i: (i, 0)),
            pl.BlockSpec((_BLK, 1), lambda i: (i, 0)),
            pl.BlockSpec((1, _F), lambda i: (0, 0)),
            pl.BlockSpec((1, 1, _BLK), lambda i: (i, 0, 0)),
            pl.BlockSpec((_F, 2), lambda i: (0, 0)),
            pl.BlockSpec((1, 2), lambda i: (0, 0)),
        ],
        out_specs=pl.BlockSpec((_G, 2), lambda i: (0, 0)),
        out_shape=jax.ShapeDtypeStruct((_G, 2), jnp.float32),
        scratch_shapes=[
            pltpu.VMEM((_G, _F), jnp.float32),
            pltpu.VMEM((_G, _F), jnp.float32),
        ],
    )(part, hs2, dis, b2r, batch3, Wlin, blr)


def kernel(x, edge_index, batch, W1, b1, gamma, beta, W2, b2, Wlin, blin):
    epw = _E // _NW
    pad = _CPW * _CH - epw
    src_r = jnp.pad(edge_index[0].reshape(_NW, epw), ((0, 0), (0, pad)),
                    constant_values=_N).reshape(_NW, _CPW, _CH)
    dst_r = jnp.pad(edge_index[1].reshape(_NW, epw), ((0, 0), (0, pad)),
                    constant_values=_N).reshape(_NW, _CPW, _CH)
    zeros128 = jnp.zeros((_RPT, _F), jnp.float32)
    ones128 = jnp.tile(
        (jnp.arange(_F, dtype=jnp.int32) == 0).astype(jnp.float32)[None, :],
        (_CH, 1))
    batch3 = batch.reshape(_NB, 1, _BLK)
    b1r = b1.reshape(1, _F)
    b2r = b2.reshape(1, _F)
    gr = gamma.reshape(1, _F)
    br = beta.reshape(1, _F)
    blr = blin.reshape(1, 2)

    degp = _sc_deg(dst_r, ones128, zeros128)
    h0 = _tc_matmul(x, W1)
    dis, hs1 = _tc_dis_prescale(degp, h0)
    part1 = _sc_msg(hs1, src_r, dst_r, zeros128)
    a, stats = _tc_layer1_post(part1, hs1, dis, b1r)
    hs2 = _tc_bn_mm(a, stats, gr, br, dis, W2)
    part2 = _sc_msg(hs2, src_r, dst_r, zeros128)
    return _tc_final(part2, hs2, dis, b2r, batch3, Wlin, blr)
